# Initial kernel scaffold; baseline (speedup 1.0000x reference)
import jax
import jax.numpy as jnp
from jax.experimental import pallas as pl


def kernel(x_nchw, conv0_w, conv0_b, conv1_w, conv1_b, conv2_w, conv2_b, conv3_w, conv3_b, conv4_w, conv4_b, conv5_w, conv5_b, conv6_w, conv6_b, conv7_w, conv7_b, conv8_w, conv8_b, conv9_w, conv9_b, conv10_w, conv10_b, conv11_w, conv11_b, conv12_w, conv12_b, fc_w1, fc_b1, fc_w2, fc_b2):
    raise NotImplementedError("write your pallas kernel here")


# speedup vs baseline: 4.8798x; 4.8798x over previous
import functools

import jax
import jax.numpy as jnp
from jax.experimental import pallas as pl
from jax.experimental.pallas import tpu as pltpu

_NCLS = 10


def _shift3(x):
    z = jnp.zeros((1, x.shape[1]), x.dtype)
    xm = jnp.concatenate([z, x[:-1]], axis=0)
    xp = jnp.concatenate([x[1:], z], axis=0)
    return xm, x, xp


def _conv_row(x_ref, w_ref, b_ref, r, cin_k):
    xw = x_ref[pl.ds(r - 1, 3)]
    parts = []
    for dy in range(3):
        xr = xw[dy]
        if cin_k < xr.shape[1]:
            xr = xr[:, :cin_k]
        parts.extend(_shift3(xr))
    xc = jnp.concatenate(parts, axis=1)
    acc = jnp.dot(xc, w_ref[...], preferred_element_type=jnp.float32)
    return jnp.maximum(acc + b_ref[...], 0.0)


def _conv_plain_kernel(x_ref, w_ref, b_ref, o_ref, *, hh, wp, cin_k):
    tm = x_ref.shape[1]
    cout = o_ref.shape[2]
    col = jax.lax.broadcasted_iota(jnp.int32, (tm, 1), 0) % wp
    valid = jnp.logical_and(col > 0, col < wp - 1)
    zrow = jnp.zeros((1, tm, cout), o_ref.dtype)
    o_ref[pl.ds(0, 1)] = zrow
    o_ref[pl.ds(hh + 1, 1)] = zrow

    def body(r, _):
        acc = _conv_row(x_ref, w_ref, b_ref, r, cin_k)
        out = jnp.where(valid, acc, 0.0).astype(o_ref.dtype)
        o_ref[pl.ds(r, 1)] = out[None]
        return _

    jax.lax.fori_loop(1, hh + 1, body, None)


def _conv_pool_kernel(x_ref, w_ref, b_ref, o_ref, *, hh, wp, cin_k, btile):
    w_valid = wp - 2
    wo = w_valid // 2
    wpn = wo + 2
    cout = o_ref.shape[2]
    tmo = o_ref.shape[1]
    ho = hh // 2
    zrow = jnp.zeros((1, tmo, cout), o_ref.dtype)
    o_ref[pl.ds(0, 1)] = zrow
    o_ref[pl.ds(ho + 1, 1)] = zrow

    def body(i, _):
        c0 = _conv_row(x_ref, w_ref, b_ref, 2 * i - 1, cin_k)
        c1 = _conv_row(x_ref, w_ref, b_ref, 2 * i, cin_k)
        v = jnp.maximum(c0, c1).reshape(btile, wp, cout)
        v = v[:, 1:1 + w_valid].reshape(btile, wo, 2, cout).max(axis=2)
        zc = jnp.zeros((btile, 1, cout), v.dtype)
        v = jnp.concatenate([zc, v, zc], axis=1).reshape(tmo, cout)
        o_ref[pl.ds(i, 1)] = v[None].astype(o_ref.dtype)
        return _

    jax.lax.fori_loop(1, ho + 1, body, None)


def _conv_fc_kernel(x_ref, w_ref, b_ref, w1_ref, b1_ref, w2_ref, b2_ref,
                    o_ref, *, wp, cin_k, btile):
    cout = w_ref.shape[1]
    c0 = _conv_row(x_ref, w_ref, b_ref, 1, cin_k)
    c1 = _conv_row(x_ref, w_ref, b_ref, 2, cin_k)
    v = jnp.maximum(c0, c1).reshape(btile, wp, cout)
    feat = v[:, 1:3].max(axis=1).astype(jnp.bfloat16)
    h = jnp.dot(feat, w1_ref[...], preferred_element_type=jnp.float32)
    h = jnp.maximum(h + b1_ref[...], 0.0).astype(jnp.bfloat16)
    logits = jnp.dot(h, w2_ref[...], preferred_element_type=jnp.float32)
    o_ref[...] = logits + b2_ref[...]


_CFG = [
    (32, 8, 8, 128, False, 16),
    (32, 128, 64, 128, True, 16),
    (16, 128, 64, 128, False, 16),
    (16, 128, 128, 128, True, 16),
    (8, 128, 128, 256, False, 32),
    (8, 256, 256, 256, False, 32),
    (8, 256, 256, 256, True, 32),
    (4, 256, 256, 512, False, 32),
    (4, 512, 512, 512, False, 32),
    (4, 512, 512, 512, True, 32),
    (2, 512, 512, 512, False, 32),
    (2, 512, 512, 512, False, 32),
    (2, 512, 512, 512, True, 32),
]

_VMEM = dict(vmem_limit_bytes=64 * 1024 * 1024)


def _conv_call(x, wcat, bias, *, hh, cin_k, cout, pool, btile):
    hp, m, _ = x.shape
    wp = hh + 2
    nb = m // wp
    btile = min(btile, nb)
    n_bt = nb // btile
    tm = btile * wp
    if pool:
        ho = hh // 2
        wo = (wp - 2) // 2
        wpn = wo + 2
        out_shape = jax.ShapeDtypeStruct((ho + 2, nb * wpn, cout), jnp.bfloat16)
        out_spec = pl.BlockSpec((ho + 2, btile * wpn, cout), lambda i: (0, i, 0))
        kern = functools.partial(_conv_pool_kernel, hh=hh, wp=wp,
                                 cin_k=cin_k, btile=btile)
    else:
        out_shape = jax.ShapeDtypeStruct((hp, m, cout), jnp.bfloat16)
        out_spec = pl.BlockSpec((hp, tm, cout), lambda i: (0, i, 0))
        kern = functools.partial(_conv_plain_kernel, hh=hh, wp=wp, cin_k=cin_k)
    return pl.pallas_call(
        kern,
        out_shape=out_shape,
        grid=(n_bt,),
        in_specs=[
            pl.BlockSpec((hp, tm, x.shape[2]), lambda i: (0, i, 0)),
            pl.BlockSpec(wcat.shape, lambda i: (0, 0)),
            pl.BlockSpec(bias.shape, lambda i: (0, 0)),
        ],
        out_specs=out_spec,
        compiler_params=pltpu.CompilerParams(
            dimension_semantics=("parallel",), **_VMEM),
    )(x, wcat, bias)


def _conv_fc_call(x, wcat, bias, w1, b1, w2, b2, *, cin_k, btile):
    hp, m, cin = x.shape
    wp = 4
    nb = m // wp
    btile = min(btile, nb)
    n_bt = nb // btile
    tm = btile * wp
    ncp = w2.shape[1]
    kern = functools.partial(_conv_fc_kernel, wp=wp, cin_k=cin_k, btile=btile)
    return pl.pallas_call(
        kern,
        out_shape=jax.ShapeDtypeStruct((nb, ncp), jnp.float32),
        grid=(n_bt,),
        in_specs=[
            pl.BlockSpec((hp, tm, cin), lambda i: (0, i, 0)),
            pl.BlockSpec(wcat.shape, lambda i: (0, 0)),
            pl.BlockSpec(bias.shape, lambda i: (0, 0)),
            pl.BlockSpec(w1.shape, lambda i: (0, 0)),
            pl.BlockSpec(b1.shape, lambda i: (0, 0)),
            pl.BlockSpec(w2.shape, lambda i: (0, 0)),
            pl.BlockSpec(b2.shape, lambda i: (0, 0)),
        ],
        out_specs=pl.BlockSpec((btile, ncp), lambda i: (i, 0)),
        compiler_params=pltpu.CompilerParams(
            dimension_semantics=("parallel",), **_VMEM),
    )(x, wcat, bias, w1, b1, w2, b2)


def _prep_x(x_nchw):
    b = x_nchw.shape[0]
    x = jnp.transpose(x_nchw, (0, 2, 3, 1)).astype(jnp.bfloat16)
    x = jnp.pad(x, ((0, 0), (0, 0), (1, 1), (0, 5)))
    x = jnp.transpose(x, (1, 0, 2, 3)).reshape(32, b * 34, 8)
    return jnp.pad(x, ((1, 1), (0, 0), (0, 0)))


def _prep_w(w, cin_k):
    if w.shape[1] == 3:
        w = jnp.pad(w, ((0, 0), (0, 5), (0, 0)))
    if cin_k < w.shape[1]:
        w = w[:, :cin_k]
    return w.reshape(9 * cin_k, w.shape[2])


def kernel(x_nchw, conv0_w, conv0_b, conv1_w, conv1_b, conv2_w, conv2_b,
           conv3_w, conv3_b, conv4_w, conv4_b, conv5_w, conv5_b,
           conv6_w, conv6_b, conv7_w, conv7_b, conv8_w, conv8_b,
           conv9_w, conv9_b, conv10_w, conv10_b, conv11_w, conv11_b,
           conv12_w, conv12_b, fc_w1, fc_b1, fc_w2, fc_b2):
    ws = [conv0_w, conv1_w, conv2_w, conv3_w, conv4_w, conv5_w, conv6_w,
          conv7_w, conv8_w, conv9_w, conv10_w, conv11_w, conv12_w]
    bs = [conv0_b, conv1_b, conv2_b, conv3_b, conv4_b, conv5_b, conv6_b,
          conv7_b, conv8_b, conv9_b, conv10_b, conv11_b, conv12_b]
    x = _prep_x(x_nchw)
    for i, (hh, _cin, cin_k, cout, pool, btile) in enumerate(_CFG):
        wcat = _prep_w(ws[i], cin_k)
        if i == len(_CFG) - 1:
            logits = _conv_fc_call(x, wcat, bs[i], fc_w1, fc_b1, fc_w2, fc_b2,
                                   cin_k=cin_k, btile=btile)
            return logits[:, :_NCLS]
        x = _conv_call(x, wcat, bs[i], hh=hh, cin_k=cin_k, cout=cout,
                       pool=pool, btile=btile)
